# Initial kernel scaffold; baseline (speedup 1.0000x reference)
import jax
import jax.numpy as jnp
from jax.experimental import pallas as pl


def kernel(x, lut):
    raise NotImplementedError("write your pallas kernel here")


# speedup vs baseline: 2.9012x; 2.9012x over previous
import functools
import math

import jax
import jax.numpy as jnp
from jax import lax
from jax.experimental import pallas as pl
from jax.experimental.pallas import tpu as pltpu
from jax.experimental.pallas import tpu_sc as plsc

_NW = 32
_G = 128
_LANES = 16


def _emb_call(B, V, D):
    b_per_w = B // _NW
    n_g = b_per_w // _G
    mesh = plsc.VectorSubcoreMesh(core_axis_name="c", subcore_axis_name="s")

    @functools.partial(
        pl.kernel,
        mesh=mesh,
        compiler_params=pltpu.CompilerParams(use_tc_tiling_on_sc=False),
        out_type=jax.ShapeDtypeStruct((B, D), jnp.float32),
        scratch_types=[
            pltpu.VMEM((n_g, _G), jnp.int32),
            pltpu.VMEM((_G, D), jnp.float32),
            pltpu.SemaphoreType.DMA,
        ],
    )
    def emb_kernel(x_hbm, lut_hbm, out_hbm, idx_v, rows_v, sem):
        scale = jnp.float32(math.sqrt(D))
        wid = lax.axis_index("s") * 2 + lax.axis_index("c")
        pltpu.sync_copy(x_hbm.at[wid], idx_v)

        def group(g, _):
            pltpu.async_copy(lut_hbm.at[idx_v.at[g]], rows_v, sem).wait()

            def srow(r, _):
                for c in range(D // _LANES):
                    sl = pl.ds(c * _LANES, _LANES)
                    rows_v[r, sl] = rows_v[r, sl] * scale
                return 0

            lax.fori_loop(0, _G, srow, 0)
            pltpu.sync_copy(rows_v, out_hbm.at[pl.ds(wid * b_per_w + g * _G, _G)])
            return 0

        lax.fori_loop(0, n_g, group, 0)

    return emb_kernel


def kernel(x, lut):
    B0, B1 = x.shape
    V, D = lut.shape
    B = B0 * B1
    x3d = x.reshape(_NW, B // (_NW * _G), _G).astype(jnp.int32)
    out = _emb_call(B, V, D)(x3d, lut)
    return out.reshape(B0, B1, D)


# speedup vs baseline: 3.4820x; 1.2002x over previous
import functools
import math

import jax
import jax.numpy as jnp
from jax import lax
from jax.experimental import pallas as pl
from jax.experimental.pallas import tpu as pltpu
from jax.experimental.pallas import tpu_sc as plsc

_NW = 32
_G = 128
_NBUF = 2
_LANES = 16


def _emb_call(B, V, D):
    b_per_w = B // _NW
    n_g = b_per_w // _G
    mesh = plsc.VectorSubcoreMesh(core_axis_name="c", subcore_axis_name="s")

    @functools.partial(
        pl.kernel,
        mesh=mesh,
        compiler_params=pltpu.CompilerParams(use_tc_tiling_on_sc=False),
        out_type=jax.ShapeDtypeStruct((B, D), jnp.float32),
        scratch_types=[
            pltpu.VMEM((b_per_w,), jnp.int32),
            pltpu.VMEM((_NBUF, _G, D), jnp.float32),
            pltpu.SemaphoreType.DMA((_NBUF,)),
            pltpu.SemaphoreType.DMA((_NBUF,)),
        ],
    )
    def emb_kernel(x_hbm, lut_hbm, out_hbm, idx_v, rows_v, in_sem, out_sem):
        scale = jnp.float32(math.sqrt(D))
        wid = lax.axis_index("s") * 2 + lax.axis_index("c")
        base = wid * b_per_w
        pltpu.sync_copy(x_hbm.at[pl.ds(base, b_per_w)], idx_v)

        def gather(g, b):
            pltpu.async_copy(
                lut_hbm.at[idx_v.at[pl.ds(g * _G, _G)]], rows_v.at[b], in_sem.at[b]
            )

        def wait_in(b):
            pltpu.make_async_copy(
                lut_hbm.at[idx_v.at[pl.ds(0, _G)]], rows_v.at[b], in_sem.at[b]
            ).wait()

        def put(g, b):
            pltpu.async_copy(
                rows_v.at[b], out_hbm.at[pl.ds(base + g * _G, _G)], out_sem.at[b]
            )

        def wait_out(b):
            pltpu.make_async_copy(
                rows_v.at[b], out_hbm.at[pl.ds(base, _G)], out_sem.at[b]
            ).wait()

        for b in range(_NBUF):
            gather(b, b)

        def outer(i, _):
            g0 = i * _NBUF
            for b in range(_NBUF):
                g = g0 + b
                wait_in(b)

                def srow(r, _):
                    for c in range(D // _LANES):
                        sl = pl.ds(c * _LANES, _LANES)
                        rows_v[b, r, sl] = rows_v[b, r, sl] * scale
                    return 0

                lax.fori_loop(0, _G, srow, 0, unroll=2)
                put(g, b)

                @pl.when(g + _NBUF < n_g)
                def _():
                    wait_out(b)
                    gather(g + _NBUF, b)

            return 0

        lax.fori_loop(0, n_g // _NBUF, outer, 0)
        for b in range(_NBUF):
            wait_out(b)

    return emb_kernel


def kernel(x, lut):
    B0, B1 = x.shape
    V, D = lut.shape
    B = B0 * B1
    x_flat = x.reshape(B).astype(jnp.int32)
    out = _emb_call(B, V, D)(x_flat, lut)
    return out.reshape(B0, B1, D)
